# baseline (speedup 1.0000x reference)
import functools

import jax
import jax.numpy as jnp
import numpy as np
from jax import lax
from jax.experimental import pallas as pl
from jax.experimental.pallas import tpu as pltpu
from jax.experimental.pallas import tpu_sc as plsc

CUT_UP = 10.0


def _lin1_body(x_ref, w_ref, o_ref):
    o_ref[...] = lax.dot_general(
        x_ref[...], w_ref[...], (((1,), (1,)), ((), ())),
        preferred_element_type=jnp.float32)


def _tc_lin1(x, lin1_w):
    n, h = x.shape
    return pl.pallas_call(
        _lin1_body,
        out_shape=jax.ShapeDtypeStruct((n, h), jnp.float32),
    )(x, lin1_w)


def _filter_body(ea_ref, ew_ref, w0_ref, b0_ref, w2_ref, b2_ref, o_ref):
    ea = ea_ref[...]
    h1 = lax.dot_general(ea, w0_ref[...], (((1,), (1,)), ((), ())),
                         preferred_element_type=jnp.float32) + b0_ref[...]
    h1 = h1 * jax.nn.sigmoid(h1)
    w = lax.dot_general(h1, w2_ref[...], (((1,), (1,)), ((), ())),
                        preferred_element_type=jnp.float32) + b2_ref[...]
    ew = ew_ref[0]
    cut = 0.5 * (jnp.cos(ew * (np.pi / CUT_UP)) + 1.0)
    cut = jnp.where(ew < CUT_UP, cut, 0.0)
    o_ref[...] = w * jnp.transpose(cut, (1, 0))


def _tc_filter(edge_attr, edge_weight, mlp_w0, mlp_b0, mlp_w2, mlp_b2):
    e, nrbf = edge_attr.shape
    nf = mlp_w0.shape[0]
    be = 2000
    grid = e // be
    ew2 = edge_weight.reshape(grid, 1, be)
    b0 = mlp_b0.reshape(1, nf)
    b2 = mlp_b2.reshape(1, nf)
    return pl.pallas_call(
        _filter_body,
        grid=(grid,),
        in_specs=[
            pl.BlockSpec((be, nrbf), lambda i: (i, 0)),
            pl.BlockSpec((1, 1, be), lambda i: (i, 0, 0)),
            pl.BlockSpec((nf, nrbf), lambda i: (0, 0)),
            pl.BlockSpec((1, nf), lambda i: (0, 0)),
            pl.BlockSpec((nf, nf), lambda i: (0, 0)),
            pl.BlockSpec((1, nf), lambda i: (0, 0)),
        ],
        out_specs=pl.BlockSpec((be, nf), lambda i: (i, 0)),
        out_shape=jax.ShapeDtypeStruct((e, nf), jnp.float32),
    )(edge_attr, ew2, mlp_w0, b0, mlp_w2, b2)


_NC = 2
_NS = 16
_L = 16


def _sc_message_passing(h, w, src, dst):
    n, hd = h.shape
    e = src.shape[0]
    nw = _NC * _NS
    epw = e // nw
    b = 40
    nchunk = epw // b
    nrows_chunks = -(-n // b)
    rounds = -(-nrows_chunks // _NS)

    mesh = plsc.VectorSubcoreMesh(core_axis_name="c", subcore_axis_name="s")

    @functools.partial(
        pl.kernel,
        mesh=mesh,
        out_type=jax.ShapeDtypeStruct((_NC, n, hd), jnp.float32),
        scratch_types=[
            pltpu.VMEM((b,), jnp.int32),
            pltpu.VMEM((b,), jnp.int32),
            pltpu.VMEM((b,), jnp.int32),
            pltpu.VMEM((b,), jnp.int32),
            pltpu.VMEM((b, hd), jnp.float32),
            pltpu.VMEM((b, hd), jnp.float32),
            pltpu.VMEM((b, hd), jnp.float32),
            pltpu.VMEM((b, hd), jnp.float32),
            pltpu.VMEM((b, hd), jnp.float32),
            pltpu.VMEM((b, hd), jnp.float32),
            pltpu.VMEM_SHARED((n, hd), jnp.float32),
            pltpu.SemaphoreType.DMA,
            pltpu.SemaphoreType.DMA,
            pltpu.SemaphoreType.DMA,
            pltpu.SemaphoreType.DMA,
            pltpu.SemaphoreType.DMA,
            pltpu.SemaphoreType.DMA,
        ],
    )
    def sc_body(h_hbm, w_hbm, src_hbm, dst_hbm, out_hbm,
                src0, src1, dst0, dst1, rows0, rows1, w0, w1, msg0, msg1,
                agg_sh, isem0, isem1, gsem0, gsem1, wsem0, wsem1):
        c = lax.axis_index("c")
        s = lax.axis_index("s")
        wid = s * _NC + c

        srcs = (src0, src1)
        dsts = (dst0, dst1)
        rows = (rows0, rows1)
        ws = (w0, w1)
        msgs = (msg0, msg1)
        isems = (isem0, isem1)
        gsems = (gsem0, gsem1)
        wsems = (wsem0, wsem1)

        zeros = jnp.zeros((_L,), jnp.float32)

        def zero_row(i, _):
            for f in range(hd // _L):
                msg0[i, pl.ds(f * _L, _L)] = zeros
            return 0

        lax.fori_loop(0, b, zero_row, 0)

        def zero_chunk(k, _):
            idx = s + k * _NS

            @pl.when(idx < nrows_chunks)
            def _z():
                pltpu.sync_copy(msg0, agg_sh.at[pl.ds(idx * b, b)])

            return 0

        lax.fori_loop(0, rounds, zero_chunk, 0)
        plsc.subcore_barrier()

        def idx_start(j, sl):
            base = wid * epw + j * b
            pltpu.async_copy(src_hbm.at[pl.ds(base, b)], srcs[sl], isems[sl])
            pltpu.async_copy(dst_hbm.at[pl.ds(base, b)], dsts[sl], isems[sl])

        def idx_wait(sl):
            pltpu.make_async_copy(src_hbm.at[pl.ds(0, b)], srcs[sl], isems[sl]).wait()
            pltpu.make_async_copy(dst_hbm.at[pl.ds(0, b)], dsts[sl], isems[sl]).wait()

        def fetch_start(j, sl):
            base = wid * epw + j * b
            pltpu.async_copy(h_hbm.at[srcs[sl]], rows[sl], gsems[sl])
            pltpu.async_copy(w_hbm.at[pl.ds(base, b)], ws[sl], wsems[sl])

        def process(sl):
            pltpu.make_async_copy(h_hbm.at[srcs[sl]], rows[sl], gsems[sl]).wait()
            pltpu.make_async_copy(w_hbm.at[pl.ds(0, b)], ws[sl], wsems[sl]).wait()
            rv = rows[sl]
            wv = ws[sl]
            mv = msgs[sl]

            def mul_body(k, _2):
                for u in range(2):
                    ei = k * 2 + u
                    for f in range(hd // _L):
                        mv[ei, pl.ds(f * _L, _L)] = (
                            rv[ei, pl.ds(f * _L, _L)]
                            * wv[ei, pl.ds(f * _L, _L)])
                return 0

            lax.fori_loop(0, b // 2, mul_body, 0)
            pltpu.sync_copy(mv, agg_sh.at[dsts[sl]], add=True)

        idx_start(0, 0)
        idx_wait(0)
        fetch_start(0, 0)
        idx_start(1, 1)

        def step(j, sl):
            other = 1 - sl

            @pl.when(j + 1 < nchunk)
            def _g():
                idx_wait(other)
                fetch_start(j + 1, other)

            process(sl)

            @pl.when(j + 2 < nchunk)
            def _i():
                idx_start(j + 2, sl)

        def pair(k, _):
            step(k * 2, 0)
            step(k * 2 + 1, 1)
            return 0

        lax.fori_loop(0, nchunk // 2, pair, 0)
        if nchunk % 2 == 1:
            step(nchunk - 1, 0)
        plsc.subcore_barrier()

        def out_chunk(k, _):
            idx = s + k * _NS

            @pl.when(idx < nrows_chunks)
            def _o():
                pltpu.sync_copy(agg_sh.at[pl.ds(idx * b, b)], msg0)
                pltpu.sync_copy(msg0, out_hbm.at[c].at[pl.ds(idx * b, b)])

            return 0

        lax.fori_loop(0, rounds, out_chunk, 0)

    return sc_body(h, w, src, dst)


def _tail_body(p_ref, w2_ref, b2_ref, wl_ref, bl_ref, o_ref):
    agg = p_ref[0] + p_ref[1]
    t = lax.dot_general(agg, w2_ref[...], (((1,), (1,)), ((), ())),
                        preferred_element_type=jnp.float32) + b2_ref[...]
    t = t * jax.nn.sigmoid(t)
    o_ref[...] = lax.dot_general(t, wl_ref[...], (((1,), (1,)), ((), ())),
                                 preferred_element_type=jnp.float32) + bl_ref[...]


def _tc_tail(partial, lin2_w, lin2_b, lin_w, lin_b):
    _, n, h = partial.shape
    bn = 2000
    grid = n // bn
    b2 = lin2_b.reshape(1, h)
    bl = lin_b.reshape(1, h)
    return pl.pallas_call(
        _tail_body,
        grid=(grid,),
        in_specs=[
            pl.BlockSpec((_NC, bn, h), lambda i: (0, i, 0)),
            pl.BlockSpec((h, h), lambda i: (0, 0)),
            pl.BlockSpec((1, h), lambda i: (0, 0)),
            pl.BlockSpec((h, h), lambda i: (0, 0)),
            pl.BlockSpec((1, h), lambda i: (0, 0)),
        ],
        out_specs=pl.BlockSpec((bn, h), lambda i: (i, 0)),
        out_shape=jax.ShapeDtypeStruct((n, h), jnp.float32),
    )(partial, lin2_w, b2, lin_w, bl)


def kernel(x, edge_index, edge_weight, edge_attr, lin1_w, lin2_w, lin2_b,
           mlp_w0, mlp_b0, mlp_w2, mlp_b2, lin_w, lin_b):
    src = edge_index[0]
    dst = edge_index[1]
    h = _tc_lin1(x, lin1_w)
    w = _tc_filter(edge_attr, edge_weight, mlp_w0, mlp_b0, mlp_w2, mlp_b2)
    partial = _sc_message_passing(h, w, src, dst)
    return _tc_tail(partial, lin2_w, lin2_b, lin_w, lin_b)


# speedup vs baseline: 1.0134x; 1.0134x over previous
import functools

import jax
import jax.numpy as jnp
import numpy as np
from jax import lax
from jax.experimental import pallas as pl
from jax.experimental.pallas import tpu as pltpu
from jax.experimental.pallas import tpu_sc as plsc

CUT_UP = 10.0


def _lin1_body(x_ref, w_ref, o_ref):
    o_ref[...] = lax.dot_general(
        x_ref[...], w_ref[...], (((1,), (1,)), ((), ())),
        preferred_element_type=jnp.float32)


def _tc_lin1(x, lin1_w):
    n, h = x.shape
    return pl.pallas_call(
        _lin1_body,
        out_shape=jax.ShapeDtypeStruct((n, h), jnp.float32),
    )(x, lin1_w)


def _filter_body(ea_ref, ew_ref, w0_ref, b0_ref, w2_ref, b2_ref, o_ref):
    ea = ea_ref[...]
    h1 = lax.dot_general(ea, w0_ref[...], (((1,), (1,)), ((), ())),
                         preferred_element_type=jnp.float32) + b0_ref[...]
    h1 = h1 * jax.nn.sigmoid(h1)
    w = lax.dot_general(h1.astype(jnp.bfloat16), w2_ref[...],
                        (((1,), (1,)), ((), ())),
                        preferred_element_type=jnp.float32) + b2_ref[...]
    ew = ew_ref[0]
    cut = 0.5 * (jnp.cos(ew * (np.pi / CUT_UP)) + 1.0)
    cut = jnp.where(ew < CUT_UP, cut, 0.0)
    o_ref[...] = w * jnp.transpose(cut, (1, 0))


def _tc_filter(edge_attr, edge_weight, mlp_w0, mlp_b0, mlp_w2, mlp_b2):
    e, nrbf = edge_attr.shape
    nf = mlp_w0.shape[0]
    be = 2000
    grid = e // be
    ew2 = edge_weight.reshape(grid, 1, be)
    b0 = mlp_b0.reshape(1, nf)
    b2 = mlp_b2.reshape(1, nf)
    return pl.pallas_call(
        _filter_body,
        grid=(grid,),
        in_specs=[
            pl.BlockSpec((be, nrbf), lambda i: (i, 0)),
            pl.BlockSpec((1, 1, be), lambda i: (i, 0, 0)),
            pl.BlockSpec((nf, nrbf), lambda i: (0, 0)),
            pl.BlockSpec((1, nf), lambda i: (0, 0)),
            pl.BlockSpec((nf, nf), lambda i: (0, 0)),
            pl.BlockSpec((1, nf), lambda i: (0, 0)),
        ],
        out_specs=pl.BlockSpec((be, nf), lambda i: (i, 0)),
        out_shape=jax.ShapeDtypeStruct((e, nf), jnp.float32),
    )(edge_attr, ew2, mlp_w0, b0, mlp_w2, b2)


_NC = 2
_NS = 16
_L = 16


def _sc_message_passing(h, w, src, dst):
    n, hd = h.shape
    e = src.shape[0]
    nw = _NC * _NS
    epw = e // nw
    b = 40
    nchunk = epw // b
    nrows_chunks = -(-n // b)
    rounds = -(-nrows_chunks // _NS)

    mesh = plsc.VectorSubcoreMesh(core_axis_name="c", subcore_axis_name="s")

    @functools.partial(
        pl.kernel,
        mesh=mesh,
        out_type=jax.ShapeDtypeStruct((_NC, n, hd), jnp.float32),
        scratch_types=[
            pltpu.VMEM((b,), jnp.int32),
            pltpu.VMEM((b,), jnp.int32),
            pltpu.VMEM((b,), jnp.int32),
            pltpu.VMEM((b,), jnp.int32),
            pltpu.VMEM((b, hd), jnp.float32),
            pltpu.VMEM((b, hd), jnp.float32),
            pltpu.VMEM((b, hd), jnp.float32),
            pltpu.VMEM((b, hd), jnp.float32),
            pltpu.VMEM((b, hd), jnp.float32),
            pltpu.VMEM((b, hd), jnp.float32),
            pltpu.VMEM_SHARED((n, hd), jnp.float32),
            pltpu.SemaphoreType.DMA,
            pltpu.SemaphoreType.DMA,
            pltpu.SemaphoreType.DMA,
            pltpu.SemaphoreType.DMA,
            pltpu.SemaphoreType.DMA,
            pltpu.SemaphoreType.DMA,
        ],
    )
    def sc_body(h_hbm, w_hbm, src_hbm, dst_hbm, out_hbm,
                src0, src1, dst0, dst1, rows0, rows1, w0, w1, msg0, msg1,
                agg_sh, isem0, isem1, gsem0, gsem1, wsem0, wsem1):
        c = lax.axis_index("c")
        s = lax.axis_index("s")
        wid = s * _NC + c

        srcs = (src0, src1)
        dsts = (dst0, dst1)
        rows = (rows0, rows1)
        ws = (w0, w1)
        msgs = (msg0, msg1)
        isems = (isem0, isem1)
        gsems = (gsem0, gsem1)
        wsems = (wsem0, wsem1)

        zeros = jnp.zeros((_L,), jnp.float32)

        def zero_row(i, _):
            for f in range(hd // _L):
                msg0[i, pl.ds(f * _L, _L)] = zeros
            return 0

        lax.fori_loop(0, b, zero_row, 0)

        def zero_chunk(k, _):
            idx = s + k * _NS

            @pl.when(idx < nrows_chunks)
            def _z():
                pltpu.sync_copy(msg0, agg_sh.at[pl.ds(idx * b, b)])

            return 0

        lax.fori_loop(0, rounds, zero_chunk, 0)
        plsc.subcore_barrier()

        def idx_start(j, sl):
            base = wid * epw + j * b
            pltpu.async_copy(src_hbm.at[pl.ds(base, b)], srcs[sl], isems[sl])
            pltpu.async_copy(dst_hbm.at[pl.ds(base, b)], dsts[sl], isems[sl])

        def idx_wait(sl):
            pltpu.make_async_copy(src_hbm.at[pl.ds(0, b)], srcs[sl], isems[sl]).wait()
            pltpu.make_async_copy(dst_hbm.at[pl.ds(0, b)], dsts[sl], isems[sl]).wait()

        def fetch_start(j, sl):
            base = wid * epw + j * b
            pltpu.async_copy(h_hbm.at[srcs[sl]], rows[sl], gsems[sl])
            pltpu.async_copy(w_hbm.at[pl.ds(base, b)], ws[sl], wsems[sl])

        def process(sl):
            pltpu.make_async_copy(h_hbm.at[srcs[sl]], rows[sl], gsems[sl]).wait()
            pltpu.make_async_copy(w_hbm.at[pl.ds(0, b)], ws[sl], wsems[sl]).wait()
            rv = rows[sl]
            wv = ws[sl]
            mv = msgs[sl]

            def mul_body(k, _2):
                for u in range(2):
                    ei = k * 2 + u
                    for f in range(hd // _L):
                        mv[ei, pl.ds(f * _L, _L)] = (
                            rv[ei, pl.ds(f * _L, _L)]
                            * wv[ei, pl.ds(f * _L, _L)])
                return 0

            lax.fori_loop(0, b // 2, mul_body, 0)
            pltpu.sync_copy(mv, agg_sh.at[dsts[sl]], add=True)

        idx_start(0, 0)
        idx_wait(0)
        fetch_start(0, 0)
        idx_start(1, 1)

        def step(j, sl):
            other = 1 - sl

            @pl.when(j + 1 < nchunk)
            def _g():
                idx_wait(other)
                fetch_start(j + 1, other)

            process(sl)

            @pl.when(j + 2 < nchunk)
            def _i():
                idx_start(j + 2, sl)

        def pair(k, _):
            step(k * 2, 0)
            step(k * 2 + 1, 1)
            return 0

        lax.fori_loop(0, nchunk // 2, pair, 0)
        if nchunk % 2 == 1:
            step(nchunk - 1, 0)
        plsc.subcore_barrier()

        def out_chunk(k, _):
            idx = s + k * _NS

            @pl.when(idx < nrows_chunks)
            def _o():
                pltpu.sync_copy(agg_sh.at[pl.ds(idx * b, b)], msg0)
                pltpu.sync_copy(msg0, out_hbm.at[c].at[pl.ds(idx * b, b)])

            return 0

        lax.fori_loop(0, rounds, out_chunk, 0)

    return sc_body(h, w, src, dst)


def _tail_body(p_ref, w2_ref, b2_ref, wl_ref, bl_ref, o_ref):
    agg = p_ref[0] + p_ref[1]
    t = lax.dot_general(agg, w2_ref[...], (((1,), (1,)), ((), ())),
                        preferred_element_type=jnp.float32) + b2_ref[...]
    t = t * jax.nn.sigmoid(t)
    o_ref[...] = lax.dot_general(t, wl_ref[...], (((1,), (1,)), ((), ())),
                                 preferred_element_type=jnp.float32) + bl_ref[...]


def _tc_tail(partial, lin2_w, lin2_b, lin_w, lin_b):
    _, n, h = partial.shape
    bn = 2000
    grid = n // bn
    b2 = lin2_b.reshape(1, h)
    bl = lin_b.reshape(1, h)
    return pl.pallas_call(
        _tail_body,
        grid=(grid,),
        in_specs=[
            pl.BlockSpec((_NC, bn, h), lambda i: (0, i, 0)),
            pl.BlockSpec((h, h), lambda i: (0, 0)),
            pl.BlockSpec((1, h), lambda i: (0, 0)),
            pl.BlockSpec((h, h), lambda i: (0, 0)),
            pl.BlockSpec((1, h), lambda i: (0, 0)),
        ],
        out_specs=pl.BlockSpec((bn, h), lambda i: (i, 0)),
        out_shape=jax.ShapeDtypeStruct((n, h), jnp.float32),
    )(partial, lin2_w, b2, lin_w, bl)


def kernel(x, edge_index, edge_weight, edge_attr, lin1_w, lin2_w, lin2_b,
           mlp_w0, mlp_b0, mlp_w2, mlp_b2, lin_w, lin_b):
    src = edge_index[0]
    dst = edge_index[1]
    h = _tc_lin1(x, lin1_w)
    w = _tc_filter(edge_attr.astype(jnp.bfloat16), edge_weight,
                   mlp_w0.astype(jnp.bfloat16), mlp_b0,
                   mlp_w2.astype(jnp.bfloat16), mlp_b2)
    partial = _sc_message_passing(h, w, src, dst)
    return _tc_tail(partial, lin2_w, lin2_b, lin_w, lin_b)


# speedup vs baseline: 1.2337x; 1.2174x over previous
import functools

import jax
import jax.numpy as jnp
import numpy as np
from jax import lax
from jax.experimental import pallas as pl
from jax.experimental.pallas import tpu as pltpu
from jax.experimental.pallas import tpu_sc as plsc

CUT_UP = 10.0


def _lin1_body(x_ref, w_ref, o_ref):
    o_ref[...] = lax.dot_general(
        x_ref[...], w_ref[...], (((1,), (1,)), ((), ())),
        preferred_element_type=jnp.float32)


def _tc_lin1(x, lin1_w):
    n, h = x.shape
    return pl.pallas_call(
        _lin1_body,
        out_shape=jax.ShapeDtypeStruct((n, h), jnp.float32),
    )(x, lin1_w)


def _filter_body(ea_ref, ew_ref, w0_ref, b0_ref, w2_ref, b2_ref, o_ref):
    ea = ea_ref[...]
    h1 = lax.dot_general(ea, w0_ref[...], (((0,), (1,)), ((), ())),
                         preferred_element_type=jnp.float32) + b0_ref[...]
    h1 = h1 * jax.nn.sigmoid(h1)
    w = lax.dot_general(h1.astype(jnp.bfloat16), w2_ref[...],
                        (((1,), (1,)), ((), ())),
                        preferred_element_type=jnp.float32) + b2_ref[...]
    ew = ew_ref[0]
    cut = 0.5 * (jnp.cos(ew * (np.pi / CUT_UP)) + 1.0)
    cut = jnp.where(ew < CUT_UP, cut, 0.0)
    o_ref[...] = w * jnp.transpose(cut, (1, 0))


def _tc_filter(edge_attr_t, edge_weight, mlp_w0, mlp_b0, mlp_w2, mlp_b2):
    nrbf, e = edge_attr_t.shape
    nf = mlp_w0.shape[0]
    be = 2560
    grid = e // be
    ew2 = edge_weight.reshape(grid, 1, be)
    b0 = mlp_b0.reshape(1, nf)
    b2 = mlp_b2.reshape(1, nf)
    return pl.pallas_call(
        _filter_body,
        grid=(grid,),
        in_specs=[
            pl.BlockSpec((nrbf, be), lambda i: (0, i)),
            pl.BlockSpec((1, 1, be), lambda i: (i, 0, 0)),
            pl.BlockSpec((nf, nrbf), lambda i: (0, 0)),
            pl.BlockSpec((1, nf), lambda i: (0, 0)),
            pl.BlockSpec((nf, nf), lambda i: (0, 0)),
            pl.BlockSpec((1, nf), lambda i: (0, 0)),
        ],
        out_specs=pl.BlockSpec((be, nf), lambda i: (i, 0)),
        out_shape=jax.ShapeDtypeStruct((e, nf), jnp.float32),
    )(edge_attr_t, ew2, mlp_w0, b0, mlp_w2, b2)


_NC = 2
_NS = 16
_L = 16


def _sc_message_passing(h, w, src, dst):
    n, hd = h.shape
    e = src.shape[0]
    nw = _NC * _NS
    epw = e // nw
    b = 40
    nchunk = epw // b
    nrows_chunks = -(-n // b)
    rounds = -(-nrows_chunks // _NS)

    mesh = plsc.VectorSubcoreMesh(core_axis_name="c", subcore_axis_name="s")

    @functools.partial(
        pl.kernel,
        mesh=mesh,
        out_type=jax.ShapeDtypeStruct((_NC, n, hd), jnp.float32),
        scratch_types=[
            pltpu.VMEM((b,), jnp.int32),
            pltpu.VMEM((b,), jnp.int32),
            pltpu.VMEM((b,), jnp.int32),
            pltpu.VMEM((b,), jnp.int32),
            pltpu.VMEM((b, hd), jnp.float32),
            pltpu.VMEM((b, hd), jnp.float32),
            pltpu.VMEM((b, hd), jnp.float32),
            pltpu.VMEM((b, hd), jnp.float32),
            pltpu.VMEM((b, hd), jnp.float32),
            pltpu.VMEM((b, hd), jnp.float32),
            pltpu.VMEM_SHARED((n, hd), jnp.float32),
            pltpu.SemaphoreType.DMA,
            pltpu.SemaphoreType.DMA,
            pltpu.SemaphoreType.DMA,
            pltpu.SemaphoreType.DMA,
            pltpu.SemaphoreType.DMA,
            pltpu.SemaphoreType.DMA,
        ],
    )
    def sc_body(h_hbm, w_hbm, src_hbm, dst_hbm, out_hbm,
                src0, src1, dst0, dst1, rows0, rows1, w0, w1, msg0, msg1,
                agg_sh, isem0, isem1, gsem0, gsem1, wsem0, wsem1):
        c = lax.axis_index("c")
        s = lax.axis_index("s")
        wid = s * _NC + c

        srcs = (src0, src1)
        dsts = (dst0, dst1)
        rows = (rows0, rows1)
        ws = (w0, w1)
        msgs = (msg0, msg1)
        isems = (isem0, isem1)
        gsems = (gsem0, gsem1)
        wsems = (wsem0, wsem1)

        zeros = jnp.zeros((_L,), jnp.float32)

        def zero_row(i, _):
            for f in range(hd // _L):
                msg0[i, pl.ds(f * _L, _L)] = zeros
            return 0

        lax.fori_loop(0, b, zero_row, 0)

        def zero_chunk(k, _):
            idx = s + k * _NS

            @pl.when(idx < nrows_chunks)
            def _z():
                pltpu.sync_copy(msg0, agg_sh.at[pl.ds(idx * b, b)])

            return 0

        lax.fori_loop(0, rounds, zero_chunk, 0)
        plsc.subcore_barrier()

        def idx_start(j, sl):
            base = wid * epw + j * b
            pltpu.async_copy(src_hbm.at[pl.ds(base, b)], srcs[sl], isems[sl])
            pltpu.async_copy(dst_hbm.at[pl.ds(base, b)], dsts[sl], isems[sl])

        def idx_wait(sl):
            pltpu.make_async_copy(src_hbm.at[pl.ds(0, b)], srcs[sl], isems[sl]).wait()
            pltpu.make_async_copy(dst_hbm.at[pl.ds(0, b)], dsts[sl], isems[sl]).wait()

        def fetch_start(j, sl):
            base = wid * epw + j * b
            pltpu.async_copy(h_hbm.at[srcs[sl]], rows[sl], gsems[sl])
            pltpu.async_copy(w_hbm.at[pl.ds(base, b)], ws[sl], wsems[sl])

        def process(sl):
            pltpu.make_async_copy(h_hbm.at[srcs[sl]], rows[sl], gsems[sl]).wait()
            pltpu.make_async_copy(w_hbm.at[pl.ds(0, b)], ws[sl], wsems[sl]).wait()
            rv = rows[sl]
            wv = ws[sl]
            mv = msgs[sl]

            def mul_body(k, _2):
                for u in range(2):
                    ei = k * 2 + u
                    for f in range(hd // _L):
                        mv[ei, pl.ds(f * _L, _L)] = (
                            rv[ei, pl.ds(f * _L, _L)]
                            * wv[ei, pl.ds(f * _L, _L)])
                return 0

            lax.fori_loop(0, b // 2, mul_body, 0)
            pltpu.sync_copy(mv, agg_sh.at[dsts[sl]], add=True)

        idx_start(0, 0)
        idx_wait(0)
        fetch_start(0, 0)
        idx_start(1, 1)

        def step(j, sl):
            other = 1 - sl

            @pl.when(j + 1 < nchunk)
            def _g():
                idx_wait(other)
                fetch_start(j + 1, other)

            process(sl)

            @pl.when(j + 2 < nchunk)
            def _i():
                idx_start(j + 2, sl)

        def pair(k, _):
            step(k * 2, 0)
            step(k * 2 + 1, 1)
            return 0

        lax.fori_loop(0, nchunk // 2, pair, 0)
        if nchunk % 2 == 1:
            step(nchunk - 1, 0)
        plsc.subcore_barrier()

        def out_chunk(k, _):
            idx = s + k * _NS

            @pl.when(idx < nrows_chunks)
            def _o():
                pltpu.sync_copy(agg_sh.at[pl.ds(idx * b, b)], msg0)
                pltpu.sync_copy(msg0, out_hbm.at[c].at[pl.ds(idx * b, b)])

            return 0

        lax.fori_loop(0, rounds, out_chunk, 0)

    return sc_body(h, w, src, dst)


def _tail_body(p_ref, w2_ref, b2_ref, wl_ref, bl_ref, o_ref):
    agg = p_ref[0] + p_ref[1]
    t = lax.dot_general(agg, w2_ref[...], (((1,), (1,)), ((), ())),
                        preferred_element_type=jnp.float32) + b2_ref[...]
    t = t * jax.nn.sigmoid(t)
    o_ref[...] = lax.dot_general(t, wl_ref[...], (((1,), (1,)), ((), ())),
                                 preferred_element_type=jnp.float32) + bl_ref[...]


def _tc_tail(partial, lin2_w, lin2_b, lin_w, lin_b):
    _, n, h = partial.shape
    bn = 2000
    grid = n // bn
    b2 = lin2_b.reshape(1, h)
    bl = lin_b.reshape(1, h)
    return pl.pallas_call(
        _tail_body,
        grid=(grid,),
        in_specs=[
            pl.BlockSpec((_NC, bn, h), lambda i: (0, i, 0)),
            pl.BlockSpec((h, h), lambda i: (0, 0)),
            pl.BlockSpec((1, h), lambda i: (0, 0)),
            pl.BlockSpec((h, h), lambda i: (0, 0)),
            pl.BlockSpec((1, h), lambda i: (0, 0)),
        ],
        out_specs=pl.BlockSpec((bn, h), lambda i: (i, 0)),
        out_shape=jax.ShapeDtypeStruct((n, h), jnp.float32),
    )(partial, lin2_w, b2, lin_w, bl)


def kernel(x, edge_index, edge_weight, edge_attr, lin1_w, lin2_w, lin2_b,
           mlp_w0, mlp_b0, mlp_w2, mlp_b2, lin_w, lin_b):
    src = edge_index[0]
    dst = edge_index[1]
    h = _tc_lin1(x, lin1_w)
    ea_t = jnp.transpose(edge_attr, (1, 0)).astype(jnp.bfloat16)
    w = _tc_filter(ea_t, edge_weight,
                   mlp_w0.astype(jnp.bfloat16), mlp_b0,
                   mlp_w2.astype(jnp.bfloat16), mlp_b2)
    partial = _sc_message_passing(h, w, src, dst)
    return _tc_tail(partial, lin2_w, lin2_b, lin_w, lin_b)


# speedup vs baseline: 1.3505x; 1.0947x over previous
import functools

import jax
import jax.numpy as jnp
import numpy as np
from jax import lax
from jax.experimental import pallas as pl
from jax.experimental.pallas import tpu as pltpu
from jax.experimental.pallas import tpu_sc as plsc

CUT_UP = 10.0


def _lin1_body(x_ref, w_ref, o_ref):
    o_ref[...] = lax.dot_general(
        x_ref[...], w_ref[...], (((1,), (1,)), ((), ())),
        preferred_element_type=jnp.float32)


def _tc_lin1(x, lin1_w):
    n, h = x.shape
    return pl.pallas_call(
        _lin1_body,
        out_shape=jax.ShapeDtypeStruct((n, h), jnp.float32),
    )(x, lin1_w)


def _filter_body(ea_ref, ew_ref, w0_ref, b0_ref, w2_ref, b2_ref, o_ref):
    ea = ea_ref[...]
    h1 = lax.dot_general(ea, w0_ref[...], (((0,), (1,)), ((), ())),
                         preferred_element_type=jnp.float32) + b0_ref[...]
    h1 = h1 * jax.nn.sigmoid(h1)
    w = lax.dot_general(h1.astype(jnp.bfloat16), w2_ref[...],
                        (((1,), (1,)), ((), ())),
                        preferred_element_type=jnp.float32) + b2_ref[...]
    ew = ew_ref[0]
    cut = 0.5 * (jnp.cos(ew * (np.pi / CUT_UP)) + 1.0)
    cut = jnp.where(ew < CUT_UP, cut, 0.0)
    o_ref[...] = w * jnp.transpose(cut, (1, 0))


_BE = 3200


def _tc_filter(edge_attr_t, ew2, mlp_w0, b0, mlp_w2, b2, e0, ecnt):
    nrbf, _ = edge_attr_t.shape
    nf = mlp_w0.shape[0]
    be = _BE
    grid = ecnt // be
    off = e0 // be
    return pl.pallas_call(
        _filter_body,
        grid=(grid,),
        in_specs=[
            pl.BlockSpec((nrbf, be), lambda i: (0, i + off)),
            pl.BlockSpec((1, 1, be), lambda i: (i + off, 0, 0)),
            pl.BlockSpec((nf, nrbf), lambda i: (0, 0)),
            pl.BlockSpec((1, nf), lambda i: (0, 0)),
            pl.BlockSpec((nf, nf), lambda i: (0, 0)),
            pl.BlockSpec((1, nf), lambda i: (0, 0)),
        ],
        out_specs=pl.BlockSpec((be, nf), lambda i: (i, 0)),
        out_shape=jax.ShapeDtypeStruct((ecnt, nf), jnp.float32),
    )(edge_attr_t, ew2, mlp_w0, b0, mlp_w2, b2)


_NC = 2
_NS = 16
_L = 16


def _sc_message_passing(h, w, src, dst, ebase):
    n, hd = h.shape
    ecnt = w.shape[0]
    nw = _NC * _NS
    epw = ecnt // nw
    b = 40
    nchunk = epw // b
    nrows_chunks = -(-n // b)
    rounds = -(-nrows_chunks // _NS)

    mesh = plsc.VectorSubcoreMesh(core_axis_name="c", subcore_axis_name="s")

    @functools.partial(
        pl.kernel,
        mesh=mesh,
        out_type=jax.ShapeDtypeStruct((_NC, n, hd), jnp.float32),
        scratch_types=[
            pltpu.VMEM((b,), jnp.int32),
            pltpu.VMEM((b,), jnp.int32),
            pltpu.VMEM((b,), jnp.int32),
            pltpu.VMEM((b,), jnp.int32),
            pltpu.VMEM((b, hd), jnp.float32),
            pltpu.VMEM((b, hd), jnp.float32),
            pltpu.VMEM((b, hd), jnp.float32),
            pltpu.VMEM((b, hd), jnp.float32),
            pltpu.VMEM((b, hd), jnp.float32),
            pltpu.VMEM((b, hd), jnp.float32),
            pltpu.VMEM_SHARED((n, hd), jnp.float32),
            pltpu.SemaphoreType.DMA,
            pltpu.SemaphoreType.DMA,
            pltpu.SemaphoreType.DMA,
            pltpu.SemaphoreType.DMA,
            pltpu.SemaphoreType.DMA,
            pltpu.SemaphoreType.DMA,
        ],
    )
    def sc_body(h_hbm, w_hbm, src_hbm, dst_hbm, out_hbm,
                src0, src1, dst0, dst1, rows0, rows1, w0, w1, msg0, msg1,
                agg_sh, isem0, isem1, gsem0, gsem1, wsem0, wsem1):
        c = lax.axis_index("c")
        s = lax.axis_index("s")
        wid = s * _NC + c

        srcs = (src0, src1)
        dsts = (dst0, dst1)
        rows = (rows0, rows1)
        ws = (w0, w1)
        msgs = (msg0, msg1)
        isems = (isem0, isem1)
        gsems = (gsem0, gsem1)
        wsems = (wsem0, wsem1)

        zeros = jnp.zeros((_L,), jnp.float32)

        def zero_row(i, _):
            for f in range(hd // _L):
                msg0[i, pl.ds(f * _L, _L)] = zeros
            return 0

        lax.fori_loop(0, b, zero_row, 0)

        def zero_chunk(k, _):
            idx = s + k * _NS

            @pl.when(idx < nrows_chunks)
            def _z():
                pltpu.sync_copy(msg0, agg_sh.at[pl.ds(idx * b, b)])

            return 0

        lax.fori_loop(0, rounds, zero_chunk, 0)
        plsc.subcore_barrier()

        def idx_start(j, sl):
            base = ebase + wid * epw + j * b
            pltpu.async_copy(src_hbm.at[pl.ds(base, b)], srcs[sl], isems[sl])
            pltpu.async_copy(dst_hbm.at[pl.ds(base, b)], dsts[sl], isems[sl])

        def idx_wait(sl):
            pltpu.make_async_copy(src_hbm.at[pl.ds(0, b)], srcs[sl], isems[sl]).wait()
            pltpu.make_async_copy(dst_hbm.at[pl.ds(0, b)], dsts[sl], isems[sl]).wait()

        def fetch_start(j, sl):
            wbase = wid * epw + j * b
            pltpu.async_copy(h_hbm.at[srcs[sl]], rows[sl], gsems[sl])
            pltpu.async_copy(w_hbm.at[pl.ds(wbase, b)], ws[sl], wsems[sl])

        def process(sl):
            pltpu.make_async_copy(h_hbm.at[srcs[sl]], rows[sl], gsems[sl]).wait()
            pltpu.make_async_copy(w_hbm.at[pl.ds(0, b)], ws[sl], wsems[sl]).wait()
            rv = rows[sl]
            wv = ws[sl]
            mv = msgs[sl]

            def mul_body(k, _2):
                for u in range(2):
                    ei = k * 2 + u
                    for f in range(hd // _L):
                        mv[ei, pl.ds(f * _L, _L)] = (
                            rv[ei, pl.ds(f * _L, _L)]
                            * wv[ei, pl.ds(f * _L, _L)])
                return 0

            lax.fori_loop(0, b // 2, mul_body, 0)
            pltpu.sync_copy(mv, agg_sh.at[dsts[sl]], add=True)

        idx_start(0, 0)
        idx_wait(0)
        fetch_start(0, 0)
        idx_start(1, 1)

        def step(j, sl):
            other = 1 - sl

            @pl.when(j + 1 < nchunk)
            def _g():
                idx_wait(other)
                fetch_start(j + 1, other)

            process(sl)

            @pl.when(j + 2 < nchunk)
            def _i():
                idx_start(j + 2, sl)

        def pair(k, _):
            step(k * 2, 0)
            step(k * 2 + 1, 1)
            return 0

        lax.fori_loop(0, nchunk // 2, pair, 0)
        if nchunk % 2 == 1:
            step(nchunk - 1, 0)
        plsc.subcore_barrier()

        def out_chunk(k, _):
            idx = s + k * _NS

            @pl.when(idx < nrows_chunks)
            def _o():
                pltpu.sync_copy(agg_sh.at[pl.ds(idx * b, b)], msg0)
                pltpu.sync_copy(msg0, out_hbm.at[c].at[pl.ds(idx * b, b)])

            return 0

        lax.fori_loop(0, rounds, out_chunk, 0)

    return sc_body(h, w, src, dst)


def _tail_body(p0_ref, p1_ref, w2_ref, b2_ref, wl_ref, bl_ref, o_ref):
    agg = (p0_ref[0] + p0_ref[1]) + (p1_ref[0] + p1_ref[1])
    t = lax.dot_general(agg, w2_ref[...], (((1,), (1,)), ((), ())),
                        preferred_element_type=jnp.float32) + b2_ref[...]
    t = t * jax.nn.sigmoid(t)
    o_ref[...] = lax.dot_general(t, wl_ref[...], (((1,), (1,)), ((), ())),
                                 preferred_element_type=jnp.float32) + bl_ref[...]


def _tc_tail(pa, pb, lin2_w, lin2_b, lin_w, lin_b):
    _, n, h = pa.shape
    bn = 2000
    grid = n // bn
    b2 = lin2_b.reshape(1, h)
    bl = lin_b.reshape(1, h)
    return pl.pallas_call(
        _tail_body,
        grid=(grid,),
        in_specs=[
            pl.BlockSpec((_NC, bn, h), lambda i: (0, i, 0)),
            pl.BlockSpec((_NC, bn, h), lambda i: (0, i, 0)),
            pl.BlockSpec((h, h), lambda i: (0, 0)),
            pl.BlockSpec((1, h), lambda i: (0, 0)),
            pl.BlockSpec((h, h), lambda i: (0, 0)),
            pl.BlockSpec((1, h), lambda i: (0, 0)),
        ],
        out_specs=pl.BlockSpec((bn, h), lambda i: (i, 0)),
        out_shape=jax.ShapeDtypeStruct((n, h), jnp.float32),
    )(pa, pb, lin2_w, b2, lin_w, bl)


def kernel(x, edge_index, edge_weight, edge_attr, lin1_w, lin2_w, lin2_b,
           mlp_w0, mlp_b0, mlp_w2, mlp_b2, lin_w, lin_b):
    src = edge_index[0]
    dst = edge_index[1]
    e = src.shape[0]
    half = e // 2
    h = _tc_lin1(x, lin1_w)
    ea_t = jnp.transpose(edge_attr, (1, 0)).astype(jnp.bfloat16)
    ew2 = edge_weight.reshape(e // _BE, 1, _BE)
    w0b = mlp_w0.astype(jnp.bfloat16)
    w2b = mlp_w2.astype(jnp.bfloat16)
    b0 = mlp_b0.reshape(1, -1)
    b2 = mlp_b2.reshape(1, -1)
    w_a = _tc_filter(ea_t, ew2, w0b, b0, w2b, b2, 0, half)
    p_a = _sc_message_passing(h, w_a, src, dst, 0)
    w_b = _tc_filter(ea_t, ew2, w0b, b0, w2b, b2, half, half)
    p_b = _sc_message_passing(h, w_b, src, dst, half)
    return _tc_tail(p_a, p_b, lin2_w, lin2_b, lin_w, lin_b)


# speedup vs baseline: 1.4055x; 1.0407x over previous
import functools

import jax
import jax.numpy as jnp
import numpy as np
from jax import lax
from jax.experimental import pallas as pl
from jax.experimental.pallas import tpu as pltpu
from jax.experimental.pallas import tpu_sc as plsc

CUT_UP = 10.0


def _lin1_body(x_ref, w_ref, o_ref):
    o_ref[...] = lax.dot_general(
        x_ref[...], w_ref[...], (((1,), (1,)), ((), ())),
        preferred_element_type=jnp.float32)


def _tc_lin1(x, lin1_w):
    n, h = x.shape
    return pl.pallas_call(
        _lin1_body,
        out_shape=jax.ShapeDtypeStruct((n, h), jnp.float32),
    )(x, lin1_w)


def _filter_body(ea_ref, ew_ref, w0_ref, b0_ref, w2_ref, b2_ref, o_ref):
    ea = ea_ref[...]
    h1 = lax.dot_general(ea, w0_ref[...], (((0,), (1,)), ((), ())),
                         preferred_element_type=jnp.float32) + b0_ref[...]
    h1 = h1 * jax.nn.sigmoid(h1)
    w = lax.dot_general(h1.astype(jnp.bfloat16), w2_ref[...],
                        (((1,), (1,)), ((), ())),
                        preferred_element_type=jnp.float32) + b2_ref[...]
    ew = ew_ref[0]
    cut = 0.5 * (jnp.cos(ew * (np.pi / CUT_UP)) + 1.0)
    cut = jnp.where(ew < CUT_UP, cut, 0.0)
    o_ref[...] = w * jnp.transpose(cut, (1, 0))


_BE = 3200


def _tc_filter(edge_attr_t, ew2, mlp_w0, b0, mlp_w2, b2, e0, ecnt):
    nrbf, _ = edge_attr_t.shape
    nf = mlp_w0.shape[0]
    be = _BE
    grid = ecnt // be
    off = e0 // be
    return pl.pallas_call(
        _filter_body,
        grid=(grid,),
        in_specs=[
            pl.BlockSpec((nrbf, be), lambda i: (0, i + off)),
            pl.BlockSpec((1, 1, be), lambda i: (i + off, 0, 0)),
            pl.BlockSpec((nf, nrbf), lambda i: (0, 0)),
            pl.BlockSpec((1, nf), lambda i: (0, 0)),
            pl.BlockSpec((nf, nf), lambda i: (0, 0)),
            pl.BlockSpec((1, nf), lambda i: (0, 0)),
        ],
        out_specs=pl.BlockSpec((be, nf), lambda i: (i, 0)),
        out_shape=jax.ShapeDtypeStruct((ecnt, nf), jnp.float32),
    )(edge_attr_t, ew2, mlp_w0, b0, mlp_w2, b2)


_NC = 2
_NS = 16
_L = 16


def _sc_message_passing(h, w, src, dst, ebase):
    n, hd = h.shape
    ecnt = w.shape[0]
    nw = _NC * _NS
    epw = ecnt // nw
    b = 40
    nchunk = epw // b
    nrows_chunks = -(-n // b)
    rounds = -(-nrows_chunks // _NS)

    mesh = plsc.VectorSubcoreMesh(core_axis_name="c", subcore_axis_name="s")

    @functools.partial(
        pl.kernel,
        mesh=mesh,
        out_type=jax.ShapeDtypeStruct((_NC, n, hd), jnp.float32),
        scratch_types=[
            pltpu.VMEM((b,), jnp.int32),
            pltpu.VMEM((b,), jnp.int32),
            pltpu.VMEM((b,), jnp.int32),
            pltpu.VMEM((b,), jnp.int32),
            pltpu.VMEM((b, hd), jnp.float32),
            pltpu.VMEM((b, hd), jnp.float32),
            pltpu.VMEM((b, hd), jnp.float32),
            pltpu.VMEM((b, hd), jnp.float32),
            pltpu.VMEM((b, hd), jnp.float32),
            pltpu.VMEM((b, hd), jnp.float32),
            pltpu.VMEM_SHARED((n, hd), jnp.float32),
            pltpu.SemaphoreType.DMA,
            pltpu.SemaphoreType.DMA,
            pltpu.SemaphoreType.DMA,
            pltpu.SemaphoreType.DMA,
            pltpu.SemaphoreType.DMA,
            pltpu.SemaphoreType.DMA,
        ],
    )
    def sc_body(h_hbm, w_hbm, src_hbm, dst_hbm, out_hbm,
                src0, src1, dst0, dst1, rows0, rows1, w0, w1, msg0, msg1,
                agg_sh, isem0, isem1, gsem0, gsem1, wsem0, wsem1):
        c = lax.axis_index("c")
        s = lax.axis_index("s")
        wid = s * _NC + c

        srcs = (src0, src1)
        dsts = (dst0, dst1)
        rows = (rows0, rows1)
        ws = (w0, w1)
        msgs = (msg0, msg1)
        isems = (isem0, isem1)
        gsems = (gsem0, gsem1)
        wsems = (wsem0, wsem1)

        zeros = jnp.zeros((_L,), jnp.float32)

        def zero_row(i, _):
            for f in range(hd // _L):
                msg0[i, pl.ds(f * _L, _L)] = zeros
            return 0

        lax.fori_loop(0, b, zero_row, 0)

        def zero_chunk(k, _):
            idx = s + k * _NS

            @pl.when(idx < nrows_chunks)
            def _z():
                pltpu.sync_copy(msg0, agg_sh.at[pl.ds(idx * b, b)])

            return 0

        lax.fori_loop(0, rounds, zero_chunk, 0)
        plsc.subcore_barrier()

        def idx_start(j, sl):
            base = ebase + wid * epw + j * b
            pltpu.async_copy(src_hbm.at[pl.ds(base, b)], srcs[sl], isems[sl])
            pltpu.async_copy(dst_hbm.at[pl.ds(base, b)], dsts[sl], isems[sl])

        def idx_wait(sl):
            pltpu.make_async_copy(src_hbm.at[pl.ds(0, b)], srcs[sl], isems[sl]).wait()
            pltpu.make_async_copy(dst_hbm.at[pl.ds(0, b)], dsts[sl], isems[sl]).wait()

        def fetch_start(j, sl):
            wbase = wid * epw + j * b
            pltpu.async_copy(h_hbm.at[srcs[sl]], rows[sl], gsems[sl])
            pltpu.async_copy(w_hbm.at[pl.ds(wbase, b)], ws[sl], wsems[sl])

        def process(sl):
            pltpu.make_async_copy(h_hbm.at[srcs[sl]], rows[sl], gsems[sl]).wait()
            pltpu.make_async_copy(w_hbm.at[pl.ds(0, b)], ws[sl], wsems[sl]).wait()
            rv = rows[sl]
            wv = ws[sl]
            mv = msgs[sl]

            def mul_body(k, _2):
                for u in range(2):
                    ei = k * 2 + u
                    for f in range(hd // _L):
                        mv[ei, pl.ds(f * _L, _L)] = (
                            rv[ei, pl.ds(f * _L, _L)]
                            * wv[ei, pl.ds(f * _L, _L)])
                return 0

            lax.fori_loop(0, b // 2, mul_body, 0)
            pltpu.sync_copy(mv, agg_sh.at[dsts[sl]], add=True)

        idx_start(0, 0)
        idx_wait(0)
        fetch_start(0, 0)
        idx_start(1, 1)

        def step(j, sl):
            other = 1 - sl

            @pl.when(j + 1 < nchunk)
            def _g():
                idx_wait(other)
                fetch_start(j + 1, other)

            process(sl)

            @pl.when(j + 2 < nchunk)
            def _i():
                idx_start(j + 2, sl)

        def pair(k, _):
            step(k * 2, 0)
            step(k * 2 + 1, 1)
            return 0

        lax.fori_loop(0, nchunk // 2, pair, 0)
        if nchunk % 2 == 1:
            step(nchunk - 1, 0)
        plsc.subcore_barrier()

        def out_chunk(k, _):
            idx = s + k * _NS

            @pl.when(idx < nrows_chunks)
            def _o():
                pltpu.sync_copy(agg_sh.at[pl.ds(idx * b, b)], msg0)
                pltpu.sync_copy(msg0, out_hbm.at[c].at[pl.ds(idx * b, b)])

            return 0

        lax.fori_loop(0, rounds, out_chunk, 0)

    return sc_body(h, w, src, dst)


def _tail_body(p0_ref, p1_ref, p2_ref, w2_ref, b2_ref, wl_ref, bl_ref, o_ref):
    agg = ((p0_ref[0] + p0_ref[1]) + (p1_ref[0] + p1_ref[1])
           + (p2_ref[0] + p2_ref[1]))
    t = lax.dot_general(agg, w2_ref[...], (((1,), (1,)), ((), ())),
                        preferred_element_type=jnp.float32) + b2_ref[...]
    t = t * jax.nn.sigmoid(t)
    o_ref[...] = lax.dot_general(t, wl_ref[...], (((1,), (1,)), ((), ())),
                                 preferred_element_type=jnp.float32) + bl_ref[...]


def _tc_tail(pa, pb, pc, lin2_w, lin2_b, lin_w, lin_b):
    _, n, h = pa.shape
    bn = 2000
    grid = n // bn
    b2 = lin2_b.reshape(1, h)
    bl = lin_b.reshape(1, h)
    return pl.pallas_call(
        _tail_body,
        grid=(grid,),
        in_specs=[
            pl.BlockSpec((_NC, bn, h), lambda i: (0, i, 0)),
            pl.BlockSpec((_NC, bn, h), lambda i: (0, i, 0)),
            pl.BlockSpec((_NC, bn, h), lambda i: (0, i, 0)),
            pl.BlockSpec((h, h), lambda i: (0, 0)),
            pl.BlockSpec((1, h), lambda i: (0, 0)),
            pl.BlockSpec((h, h), lambda i: (0, 0)),
            pl.BlockSpec((1, h), lambda i: (0, 0)),
        ],
        out_specs=pl.BlockSpec((bn, h), lambda i: (i, 0)),
        out_shape=jax.ShapeDtypeStruct((n, h), jnp.float32),
    )(pa, pb, pc, lin2_w, b2, lin_w, bl)


def kernel(x, edge_index, edge_weight, edge_attr, lin1_w, lin2_w, lin2_b,
           mlp_w0, mlp_b0, mlp_w2, mlp_b2, lin_w, lin_b):
    src = edge_index[0]
    dst = edge_index[1]
    e = src.shape[0]
    h = _tc_lin1(x, lin1_w)
    ea_t = jnp.transpose(edge_attr, (1, 0)).astype(jnp.bfloat16)
    ew2 = edge_weight.reshape(e // _BE, 1, _BE)
    w0b = mlp_w0.astype(jnp.bfloat16)
    w2b = mlp_w2.astype(jnp.bfloat16)
    b0 = mlp_b0.reshape(1, -1)
    b2 = mlp_b2.reshape(1, -1)
    s0 = e // 5
    s1 = (e - s0) // 2
    segs = ((0, s0), (s0, s1), (s0 + s1, s1))
    parts = []
    for e0, ec in segs:
        w_i = _tc_filter(ea_t, ew2, w0b, b0, w2b, b2, e0, ec)
        parts.append(_sc_message_passing(h, w_i, src, dst, e0))
    return _tc_tail(parts[0], parts[1], parts[2], lin2_w, lin2_b, lin_w,
                    lin_b)
